# Initial kernel scaffold; baseline (speedup 1.0000x reference)
import jax
import jax.numpy as jnp
from jax.experimental import pallas as pl


def kernel(x, edge_index, W1, att_src1, att_dst1, b1, W2, att_src2, att_dst2, b2):
    raise NotImplementedError("write your pallas kernel here")


# speedup vs baseline: 19.5094x; 19.5094x over previous
import functools

import jax
import jax.numpy as jnp
from jax import lax
from jax.experimental import pallas as pl
from jax.experimental.pallas import tpu as pltpu
from jax.experimental.pallas import tpu_sc as plsc

N = 10000
E = 160000
D_IN = 256
F = 64
SW = 16

NPAD = 10240
ROWBLK = 512
NROWBLK = NPAD // ROWBLK

NW = 32
EBLK = 128
NBLK = 40
EPAD = NW * NBLK * EBLK
TSLICE = NPAD // 16


def _prep1_body(x_ref, w1_ref, as_ref, ad_ref, ph_ref, q_ref, init_ref):
    h = jnp.dot(x_ref[...], w1_ref[...], preferred_element_type=jnp.float32)
    p = jnp.dot(h, as_ref[...], preferred_element_type=jnp.float32)
    q = jnp.dot(h, ad_ref[...], preferred_element_type=jnp.float32)
    a = p + q
    winit = jnp.exp(jnp.where(a >= 0, a, 0.2 * a))
    ph_ref[...] = jnp.concatenate([p, h], axis=1)
    q_ref[...] = q
    init_ref[...] = jnp.concatenate([winit * h, winit[:, :SW]], axis=1)


def _prep1(xp, W1, As64, Ad64):
    return pl.pallas_call(
        _prep1_body,
        grid=(NROWBLK,),
        in_specs=[
            pl.BlockSpec((ROWBLK, D_IN), lambda i: (i, 0)),
            pl.BlockSpec((D_IN, F), lambda i: (0, 0)),
            pl.BlockSpec((F, F), lambda i: (0, 0)),
            pl.BlockSpec((F, F), lambda i: (0, 0)),
        ],
        out_specs=[
            pl.BlockSpec((ROWBLK, 2 * F), lambda i: (i, 0)),
            pl.BlockSpec((ROWBLK, F), lambda i: (i, 0)),
            pl.BlockSpec((ROWBLK, F + SW), lambda i: (i, 0)),
        ],
        out_shape=[
            jax.ShapeDtypeStruct((NPAD, 2 * F), jnp.float32),
            jax.ShapeDtypeStruct((NPAD, F), jnp.float32),
            jax.ShapeDtypeStruct((NPAD, F + SW), jnp.float32),
        ],
    )(xp, W1, As64, Ad64)


def _mid_body(racc_ref, rs_ref, init_ref, b1_ref, w2_ref, as_ref, ad_ref,
              ph_ref, q_ref, init2_ref):
    acc = racc_ref[0] + racc_ref[1] + init_ref[:, :F]
    s16 = rs_ref[0] + rs_ref[1] + init_ref[:, F:]
    s = jnp.concatenate([s16, s16, s16, s16], axis=1)
    o = acc / s + b1_ref[...]
    h2 = jnp.where(o > 0, o, jnp.exp(jnp.minimum(o, 0.0)) - 1.0)
    g = jnp.dot(h2, w2_ref[...], preferred_element_type=jnp.float32)
    p = jnp.dot(g, as_ref[...], preferred_element_type=jnp.float32)
    q = jnp.dot(g, ad_ref[...], preferred_element_type=jnp.float32)
    a = p + q
    winit = jnp.exp(jnp.where(a >= 0, a, 0.2 * a))
    ph_ref[...] = jnp.concatenate([p, g], axis=1)
    q_ref[...] = q
    init2_ref[...] = jnp.concatenate([winit * g, winit[:, :SW]], axis=1)


def _mid(racc, rs, init1, b1, W2, As2, Ad2):
    return pl.pallas_call(
        _mid_body,
        grid=(NROWBLK,),
        in_specs=[
            pl.BlockSpec((2, ROWBLK, F), lambda i: (0, i, 0)),
            pl.BlockSpec((2, ROWBLK, SW), lambda i: (0, i, 0)),
            pl.BlockSpec((ROWBLK, F + SW), lambda i: (i, 0)),
            pl.BlockSpec((1, F), lambda i: (0, 0)),
            pl.BlockSpec((F, F), lambda i: (0, 0)),
            pl.BlockSpec((F, F), lambda i: (0, 0)),
            pl.BlockSpec((F, F), lambda i: (0, 0)),
        ],
        out_specs=[
            pl.BlockSpec((ROWBLK, 2 * F), lambda i: (i, 0)),
            pl.BlockSpec((ROWBLK, F), lambda i: (i, 0)),
            pl.BlockSpec((ROWBLK, F + SW), lambda i: (i, 0)),
        ],
        out_shape=[
            jax.ShapeDtypeStruct((NPAD, 2 * F), jnp.float32),
            jax.ShapeDtypeStruct((NPAD, F), jnp.float32),
            jax.ShapeDtypeStruct((NPAD, F + SW), jnp.float32),
        ],
    )(racc, rs, init1, b1, W2, As2, Ad2)


def _final_body(racc_ref, rs_ref, init_ref, b2_ref, out_ref):
    acc = racc_ref[0] + racc_ref[1] + init_ref[:, :F]
    s16 = rs_ref[0] + rs_ref[1] + init_ref[:, F:]
    s = jnp.concatenate([s16, s16, s16, s16], axis=1)
    o = acc / s + b2_ref[...]
    m = jnp.max(o, axis=1, keepdims=True)
    z = o - m
    lse = jnp.log(jnp.sum(jnp.exp(z), axis=1, keepdims=True))
    out_ref[...] = z - lse


def _final(racc, rs, init2, b2):
    return pl.pallas_call(
        _final_body,
        grid=(NROWBLK,),
        in_specs=[
            pl.BlockSpec((2, ROWBLK, F), lambda i: (0, i, 0)),
            pl.BlockSpec((2, ROWBLK, SW), lambda i: (0, i, 0)),
            pl.BlockSpec((ROWBLK, F + SW), lambda i: (i, 0)),
            pl.BlockSpec((1, F), lambda i: (0, 0)),
        ],
        out_specs=pl.BlockSpec((ROWBLK, F), lambda i: (i, 0)),
        out_shape=jax.ShapeDtypeStruct((NPAD, F), jnp.float32),
    )(racc, rs, init2, b2)


def _edge_body(ph_hbm, q_hbm, src_hbm, dst_hbm, zero_hbm, zero16_hbm,
               oacc_hbm, os_hbm,
               srcv, dstv, phb, qb, wb, mb, racc, rs):
    c = lax.axis_index("c")
    t = lax.axis_index("s")
    wid = c * 16 + t

    pltpu.sync_copy(src_hbm.at[wid], srcv)
    pltpu.sync_copy(dst_hbm.at[wid], dstv)
    pltpu.sync_copy(zero_hbm, racc.at[pl.ds(t * TSLICE, TSLICE)])
    pltpu.sync_copy(zero16_hbm, rs.at[pl.ds(t * TSLICE, TSLICE)])
    plsc.subcore_barrier()

    def blk(j, carry):
        pltpu.sync_copy(ph_hbm.at[srcv.at[j]], phb)
        pltpu.sync_copy(q_hbm.at[dstv.at[j]], qb)

        def comp(i, carry2):
            e = i // 4
            k = (i % 4) * 16
            p = phb[e, pl.ds(k, 16)]
            h = phb[e, pl.ds(F + k, 16)]
            q = qb[e, pl.ds(k, 16)]
            a = p + q
            w = jnp.exp(jnp.where(a >= 0, a, 0.2 * a))
            mb[e, pl.ds(k, 16)] = w * h

            @pl.when(k == 0)
            def _():
                wb[e, pl.ds(0, 16)] = w

            return carry2

        lax.fori_loop(0, EBLK * 4, comp, 0, unroll=4)

        pltpu.sync_copy(mb, racc.at[dstv.at[j]], add=True)
        pltpu.sync_copy(wb, rs.at[dstv.at[j]], add=True)
        return carry

    lax.fori_loop(0, NBLK, blk, 0)
    plsc.subcore_barrier()

    pltpu.sync_copy(racc.at[pl.ds(t * TSLICE, TSLICE)],
                    oacc_hbm.at[c, pl.ds(t * TSLICE, TSLICE)])
    pltpu.sync_copy(rs.at[pl.ds(t * TSLICE, TSLICE)],
                    os_hbm.at[c, pl.ds(t * TSLICE, TSLICE)])


@functools.cache
def _make_edge_pass():
    return pl.kernel(
        _edge_body,
        out_type=[
            jax.ShapeDtypeStruct((2, NPAD, F), jnp.float32),
            jax.ShapeDtypeStruct((2, NPAD, SW), jnp.float32),
        ],
        mesh=plsc.VectorSubcoreMesh(core_axis_name="c", subcore_axis_name="s"),
        compiler_params=pltpu.CompilerParams(use_tc_tiling_on_sc=False),
        scratch_types=[
            pltpu.VMEM((NBLK, EBLK), jnp.int32),
            pltpu.VMEM((NBLK, EBLK), jnp.int32),
            pltpu.VMEM((EBLK, 2 * F), jnp.float32),
            pltpu.VMEM((EBLK, F), jnp.float32),
            pltpu.VMEM((EBLK, SW), jnp.float32),
            pltpu.VMEM((EBLK, F), jnp.float32),
            pltpu.VMEM_SHARED((NPAD, F), jnp.float32),
            pltpu.VMEM_SHARED((NPAD, SW), jnp.float32),
        ],
    )


def _edge_pass(ph, q, src, dst, zero, zero16):
    return _make_edge_pass()(ph, q, src, dst, zero, zero16)


def _expand_heads(att, heads, ch):
    a = att.reshape(heads, ch)
    cols = jnp.arange(heads * ch)
    hd_of_col = cols % heads
    ch_of_col = cols // heads
    m = (a[hd_of_col, ch_of_col][:, None]
         * (hd_of_col[:, None] == hd_of_col[None, :]).astype(a.dtype))
    return m


def kernel(x, edge_index, W1, att_src1, att_dst1, b1, W2, att_src2,
           att_dst2, b2):
    perm = (jnp.arange(F) % 8) * 8 + jnp.arange(F) // 8
    W1p = W1[:, perm]
    b1p = b1[perm]
    W2p = W2[perm, :]

    xp = jnp.zeros((NPAD, D_IN), jnp.float32).at[:N].set(x)
    As1 = _expand_heads(att_src1, 8, 8)
    Ad1 = _expand_heads(att_dst1, 8, 8)
    As2 = _expand_heads(att_src2, 1, F)
    Ad2 = _expand_heads(att_dst2, 1, F)

    pad_ids = jnp.full((EPAD - E,), N, dtype=jnp.int32)
    src = jnp.concatenate([edge_index[0], pad_ids]).reshape(NW, NBLK, EBLK)
    dst = jnp.concatenate([edge_index[1], pad_ids]).reshape(NW, NBLK, EBLK)
    zero = jnp.zeros((TSLICE, F), jnp.float32)
    zero16 = jnp.zeros((TSLICE, SW), jnp.float32)

    ph1, q1, init1 = _prep1(xp, W1p, As1, Ad1)
    racc1, rs1 = _edge_pass(ph1, q1, src, dst, zero, zero16)
    ph2, q2, init2 = _mid(racc1, rs1, init1, b1p.reshape(1, F), W2p, As2, Ad2)
    racc2, rs2 = _edge_pass(ph2, q2, src, dst, zero, zero16)
    out = _final(racc2, rs2, init2, b2.reshape(1, F))
    return out[:N]


# speedup vs baseline: 34.0574x; 1.7457x over previous
import functools

import jax
import jax.numpy as jnp
from jax import lax
from jax.experimental import pallas as pl
from jax.experimental.pallas import tpu as pltpu
from jax.experimental.pallas import tpu_sc as plsc

N = 10000
E = 160000
D_IN = 256
F = 64
SW = 16

NPAD = 10240
ROWBLK = 512
NROWBLK = NPAD // ROWBLK

NW = 32
EBLK = 128
NBLK = 40
EPAD = NW * NBLK * EBLK
TSLICE = NPAD // 16


def _prep1_body(x_ref, w1_ref, as_ref, ad_ref, ph_ref, q_ref, init_ref):
    h = jnp.dot(x_ref[...], w1_ref[...], preferred_element_type=jnp.float32)
    p = jnp.dot(h, as_ref[...], preferred_element_type=jnp.float32)
    q = jnp.dot(h, ad_ref[...], preferred_element_type=jnp.float32)
    a = p + q
    winit = jnp.exp(jnp.where(a >= 0, a, 0.2 * a))
    ph_ref[...] = jnp.concatenate([p, h], axis=1)
    q_ref[...] = q
    init_ref[...] = jnp.concatenate([winit * h, winit[:, :SW]], axis=1)


def _prep1(xp, W1, As64, Ad64):
    return pl.pallas_call(
        _prep1_body,
        grid=(NROWBLK,),
        in_specs=[
            pl.BlockSpec((ROWBLK, D_IN), lambda i: (i, 0)),
            pl.BlockSpec((D_IN, F), lambda i: (0, 0)),
            pl.BlockSpec((F, F), lambda i: (0, 0)),
            pl.BlockSpec((F, F), lambda i: (0, 0)),
        ],
        out_specs=[
            pl.BlockSpec((ROWBLK, 2 * F), lambda i: (i, 0)),
            pl.BlockSpec((ROWBLK, F), lambda i: (i, 0)),
            pl.BlockSpec((ROWBLK, F + SW), lambda i: (i, 0)),
        ],
        out_shape=[
            jax.ShapeDtypeStruct((NPAD, 2 * F), jnp.float32),
            jax.ShapeDtypeStruct((NPAD, F), jnp.float32),
            jax.ShapeDtypeStruct((NPAD, F + SW), jnp.float32),
        ],
    )(xp, W1, As64, Ad64)


def _mid_body(racc_ref, rs_ref, init_ref, b1_ref, w2_ref, as_ref, ad_ref,
              ph_ref, q_ref, init2_ref):
    acc = racc_ref[0] + racc_ref[1] + init_ref[:, :F]
    s16 = rs_ref[0] + rs_ref[1] + init_ref[:, F:]
    s = jnp.concatenate([s16, s16, s16, s16], axis=1)
    o = acc / s + b1_ref[...]
    h2 = jnp.where(o > 0, o, jnp.exp(jnp.minimum(o, 0.0)) - 1.0)
    g = jnp.dot(h2, w2_ref[...], preferred_element_type=jnp.float32)
    p = jnp.dot(g, as_ref[...], preferred_element_type=jnp.float32)
    q = jnp.dot(g, ad_ref[...], preferred_element_type=jnp.float32)
    a = p + q
    winit = jnp.exp(jnp.where(a >= 0, a, 0.2 * a))
    ph_ref[...] = jnp.concatenate([p, g], axis=1)
    q_ref[...] = q
    init2_ref[...] = jnp.concatenate([winit * g, winit[:, :SW]], axis=1)


def _mid(racc, rs, init1, b1, W2, As2, Ad2):
    return pl.pallas_call(
        _mid_body,
        grid=(NROWBLK,),
        in_specs=[
            pl.BlockSpec((2, ROWBLK, F), lambda i: (0, i, 0)),
            pl.BlockSpec((2, ROWBLK, SW), lambda i: (0, i, 0)),
            pl.BlockSpec((ROWBLK, F + SW), lambda i: (i, 0)),
            pl.BlockSpec((1, F), lambda i: (0, 0)),
            pl.BlockSpec((F, F), lambda i: (0, 0)),
            pl.BlockSpec((F, F), lambda i: (0, 0)),
            pl.BlockSpec((F, F), lambda i: (0, 0)),
        ],
        out_specs=[
            pl.BlockSpec((ROWBLK, 2 * F), lambda i: (i, 0)),
            pl.BlockSpec((ROWBLK, F), lambda i: (i, 0)),
            pl.BlockSpec((ROWBLK, F + SW), lambda i: (i, 0)),
        ],
        out_shape=[
            jax.ShapeDtypeStruct((NPAD, 2 * F), jnp.float32),
            jax.ShapeDtypeStruct((NPAD, F), jnp.float32),
            jax.ShapeDtypeStruct((NPAD, F + SW), jnp.float32),
        ],
    )(racc, rs, init1, b1, W2, As2, Ad2)


def _final_body(racc_ref, rs_ref, init_ref, b2_ref, out_ref):
    acc = racc_ref[0] + racc_ref[1] + init_ref[:, :F]
    s16 = rs_ref[0] + rs_ref[1] + init_ref[:, F:]
    s = jnp.concatenate([s16, s16, s16, s16], axis=1)
    o = acc / s + b2_ref[...]
    m = jnp.max(o, axis=1, keepdims=True)
    z = o - m
    lse = jnp.log(jnp.sum(jnp.exp(z), axis=1, keepdims=True))
    out_ref[...] = z - lse


def _final(racc, rs, init2, b2):
    return pl.pallas_call(
        _final_body,
        grid=(NROWBLK,),
        in_specs=[
            pl.BlockSpec((2, ROWBLK, F), lambda i: (0, i, 0)),
            pl.BlockSpec((2, ROWBLK, SW), lambda i: (0, i, 0)),
            pl.BlockSpec((ROWBLK, F + SW), lambda i: (i, 0)),
            pl.BlockSpec((1, F), lambda i: (0, 0)),
        ],
        out_specs=pl.BlockSpec((ROWBLK, F), lambda i: (i, 0)),
        out_shape=jax.ShapeDtypeStruct((NPAD, F), jnp.float32),
    )(racc, rs, init2, b2)


def _edge_body(ph_hbm, q_hbm, src_hbm, dst_hbm, zero_hbm, zero16_hbm,
               oacc_hbm, os_hbm,
               srcv, dstv, phb0, phb1, qb0, qb1, wb0, wb1, mb0, mb1,
               racc, rs, gsem0, gsem1, ssem0, ssem1):
    c = lax.axis_index("c")
    t = lax.axis_index("s")
    wid = c * 16 + t
    phb = (phb0, phb1)
    qb = (qb0, qb1)
    wb = (wb0, wb1)
    mb = (mb0, mb1)
    gsem = (gsem0, gsem1)
    ssem = (ssem0, ssem1)

    pltpu.sync_copy(src_hbm.at[wid], srcv)
    pltpu.sync_copy(dst_hbm.at[wid], dstv)
    pltpu.sync_copy(zero_hbm, racc.at[pl.ds(t * TSLICE, TSLICE)])
    pltpu.sync_copy(zero16_hbm, rs.at[pl.ds(t * TSLICE, TSLICE)])

    def gather_start(j, par):
        pltpu.async_copy(ph_hbm.at[srcv.at[j]], phb[par], gsem[par])
        pltpu.async_copy(q_hbm.at[dstv.at[j]], qb[par], gsem[par])

    def gather_wait(j, par):
        pltpu.make_async_copy(ph_hbm.at[srcv.at[j]], phb[par], gsem[par]).wait()
        pltpu.make_async_copy(q_hbm.at[dstv.at[j]], qb[par], gsem[par]).wait()

    def scatter_start(j, par):
        pltpu.async_copy(mb[par], racc.at[dstv.at[j]], ssem[par], add=True)
        pltpu.async_copy(wb[par], rs.at[dstv.at[j]], ssem[par], add=True)

    def scatter_wait(j, par):
        pltpu.make_async_copy(mb[par], racc.at[dstv.at[j]], ssem[par]).wait()
        pltpu.make_async_copy(wb[par], rs.at[dstv.at[j]], ssem[par]).wait()

    def compute(par):
        pb = phb[par]
        qq = qb[par]

        def comp(e, carry2):
            for j in range(4):
                k = j * 16
                p = pb[e, pl.ds(k, 16)]
                h = pb[e, pl.ds(F + k, 16)]
                q = qq[e, pl.ds(k, 16)]
                a = p + q
                w = jnp.exp(jnp.where(a >= 0, a, 0.2 * a))
                mb[par][e, pl.ds(k, 16)] = w * h
                if j == 0:
                    wb[par][e, pl.ds(0, 16)] = w
            return carry2

        lax.fori_loop(0, EBLK, comp, 0, unroll=2)

    gather_start(0, 0)
    gather_start(1, 1)

    def blk(jj, carry):
        for par in range(2):
            j = 2 * jj + par
            gather_wait(j, par)

            @pl.when(j >= 2)
            def _():
                scatter_wait(j - 2, par)

            compute(par)
            scatter_start(j, par)

            @pl.when(j + 2 < NBLK)
            def _():
                gather_start(j + 2, par)

        return carry

    lax.fori_loop(0, NBLK // 2, blk, 0)
    scatter_wait(NBLK - 2, 0)
    scatter_wait(NBLK - 1, 1)
    plsc.subcore_barrier()

    pltpu.sync_copy(racc.at[pl.ds(t * TSLICE, TSLICE)],
                    oacc_hbm.at[c, pl.ds(t * TSLICE, TSLICE)])
    pltpu.sync_copy(rs.at[pl.ds(t * TSLICE, TSLICE)],
                    os_hbm.at[c, pl.ds(t * TSLICE, TSLICE)])


@functools.cache
def _make_edge_pass():
    return pl.kernel(
        _edge_body,
        out_type=[
            jax.ShapeDtypeStruct((2, NPAD, F), jnp.float32),
            jax.ShapeDtypeStruct((2, NPAD, SW), jnp.float32),
        ],
        mesh=plsc.VectorSubcoreMesh(core_axis_name="c", subcore_axis_name="s"),
        compiler_params=pltpu.CompilerParams(use_tc_tiling_on_sc=False),
        scratch_types=[
            pltpu.VMEM((NBLK, EBLK), jnp.int32),
            pltpu.VMEM((NBLK, EBLK), jnp.int32),
            pltpu.VMEM((EBLK, 2 * F), jnp.float32),
            pltpu.VMEM((EBLK, 2 * F), jnp.float32),
            pltpu.VMEM((EBLK, F), jnp.float32),
            pltpu.VMEM((EBLK, F), jnp.float32),
            pltpu.VMEM((EBLK, SW), jnp.float32),
            pltpu.VMEM((EBLK, SW), jnp.float32),
            pltpu.VMEM((EBLK, F), jnp.float32),
            pltpu.VMEM((EBLK, F), jnp.float32),
            pltpu.VMEM_SHARED((NPAD, F), jnp.float32),
            pltpu.VMEM_SHARED((NPAD, SW), jnp.float32),
            pltpu.SemaphoreType.DMA,
            pltpu.SemaphoreType.DMA,
            pltpu.SemaphoreType.DMA,
            pltpu.SemaphoreType.DMA,
        ],
    )


def _edge_pass(ph, q, src, dst, zero, zero16):
    return _make_edge_pass()(ph, q, src, dst, zero, zero16)


def _expand_heads(att, heads, ch):
    a = att.reshape(heads, ch)
    cols = jnp.arange(heads * ch)
    hd_of_col = cols % heads
    ch_of_col = cols // heads
    m = (a[hd_of_col, ch_of_col][:, None]
         * (hd_of_col[:, None] == hd_of_col[None, :]).astype(a.dtype))
    return m


def kernel(x, edge_index, W1, att_src1, att_dst1, b1, W2, att_src2,
           att_dst2, b2):
    perm = (jnp.arange(F) % 8) * 8 + jnp.arange(F) // 8
    W1p = W1[:, perm]
    b1p = b1[perm]
    W2p = W2[perm, :]

    xp = jnp.zeros((NPAD, D_IN), jnp.float32).at[:N].set(x)
    As1 = _expand_heads(att_src1, 8, 8)
    Ad1 = _expand_heads(att_dst1, 8, 8)
    As2 = _expand_heads(att_src2, 1, F)
    Ad2 = _expand_heads(att_dst2, 1, F)

    pad_ids = jnp.full((EPAD - E,), N, dtype=jnp.int32)
    src = jnp.concatenate([edge_index[0], pad_ids]).reshape(NW, NBLK, EBLK)
    dst = jnp.concatenate([edge_index[1], pad_ids]).reshape(NW, NBLK, EBLK)
    zero = jnp.zeros((TSLICE, F), jnp.float32)
    zero16 = jnp.zeros((TSLICE, SW), jnp.float32)

    ph1, q1, init1 = _prep1(xp, W1p, As1, Ad1)
    racc1, rs1 = _edge_pass(ph1, q1, src, dst, zero, zero16)
    ph2, q2, init2 = _mid(racc1, rs1, init1, b1p.reshape(1, F), W2p, As2, Ad2)
    racc2, rs2 = _edge_pass(ph2, q2, src, dst, zero, zero16)
    out = _final(racc2, rs2, init2, b2.reshape(1, F))
    return out[:N]


# speedup vs baseline: 43.5441x; 1.2785x over previous
import functools

import jax
import jax.numpy as jnp
from jax import lax
from jax.experimental import pallas as pl
from jax.experimental.pallas import tpu as pltpu
from jax.experimental.pallas import tpu_sc as plsc

N = 10000
E = 160000
D_IN = 256
F = 64
SW = 16
PW = SW + F

NPAD = 10240
ROWBLK = 512
NROWBLK = NPAD // ROWBLK

NW = 32
EBLK = 128
NBLK = 40
EPAD = NW * NBLK * EBLK
TSLICE = NPAD // 16


def _prep1_body(x_ref, w1_ref, as_ref, ad_ref, ph_ref, q_ref, init_ref):
    h = jnp.dot(x_ref[...], w1_ref[...], preferred_element_type=jnp.float32)
    p = jnp.dot(h, as_ref[...], preferred_element_type=jnp.float32)
    q = jnp.dot(h, ad_ref[...], preferred_element_type=jnp.float32)
    a = p + q
    winit = jnp.exp(jnp.where(a >= 0, a, 0.2 * a))
    winit64 = jnp.concatenate([winit, winit, winit, winit], axis=1)
    ph_ref[...] = jnp.concatenate([p, h], axis=1)
    q_ref[...] = q
    init_ref[...] = jnp.concatenate([winit64 * h, winit], axis=1)


def _prep1(xp, W1, As16, Ad16):
    return pl.pallas_call(
        _prep1_body,
        grid=(NROWBLK,),
        in_specs=[
            pl.BlockSpec((ROWBLK, D_IN), lambda i: (i, 0)),
            pl.BlockSpec((D_IN, F), lambda i: (0, 0)),
            pl.BlockSpec((F, SW), lambda i: (0, 0)),
            pl.BlockSpec((F, SW), lambda i: (0, 0)),
        ],
        out_specs=[
            pl.BlockSpec((ROWBLK, PW), lambda i: (i, 0)),
            pl.BlockSpec((ROWBLK, SW), lambda i: (i, 0)),
            pl.BlockSpec((ROWBLK, PW), lambda i: (i, 0)),
        ],
        out_shape=[
            jax.ShapeDtypeStruct((NPAD, PW), jnp.float32),
            jax.ShapeDtypeStruct((NPAD, SW), jnp.float32),
            jax.ShapeDtypeStruct((NPAD, PW), jnp.float32),
        ],
    )(xp, W1, As16, Ad16)


def _mid_body(r_ref, init_ref, b1_ref, w2_ref, as_ref, ad_ref,
              ph_ref, q_ref, init2_ref):
    acc = r_ref[0, :, :F] + r_ref[1, :, :F] + init_ref[:, :F]
    s16 = r_ref[0, :, F:] + r_ref[1, :, F:] + init_ref[:, F:]
    s = jnp.concatenate([s16, s16, s16, s16], axis=1)
    o = acc / s + b1_ref[...]
    h2 = jnp.where(o > 0, o, jnp.exp(jnp.minimum(o, 0.0)) - 1.0)
    g = jnp.dot(h2, w2_ref[...], preferred_element_type=jnp.float32)
    p = jnp.dot(g, as_ref[...], preferred_element_type=jnp.float32)
    q = jnp.dot(g, ad_ref[...], preferred_element_type=jnp.float32)
    a = p + q
    winit = jnp.exp(jnp.where(a >= 0, a, 0.2 * a))
    winit64 = jnp.concatenate([winit, winit, winit, winit], axis=1)
    ph_ref[...] = jnp.concatenate([p, g], axis=1)
    q_ref[...] = q
    init2_ref[...] = jnp.concatenate([winit64 * g, winit], axis=1)


def _mid(r, init1, b1, W2, As2, Ad2):
    return pl.pallas_call(
        _mid_body,
        grid=(NROWBLK,),
        in_specs=[
            pl.BlockSpec((2, ROWBLK, PW), lambda i: (0, i, 0)),
            pl.BlockSpec((ROWBLK, PW), lambda i: (i, 0)),
            pl.BlockSpec((1, F), lambda i: (0, 0)),
            pl.BlockSpec((F, F), lambda i: (0, 0)),
            pl.BlockSpec((F, SW), lambda i: (0, 0)),
            pl.BlockSpec((F, SW), lambda i: (0, 0)),
        ],
        out_specs=[
            pl.BlockSpec((ROWBLK, PW), lambda i: (i, 0)),
            pl.BlockSpec((ROWBLK, SW), lambda i: (i, 0)),
            pl.BlockSpec((ROWBLK, PW), lambda i: (i, 0)),
        ],
        out_shape=[
            jax.ShapeDtypeStruct((NPAD, PW), jnp.float32),
            jax.ShapeDtypeStruct((NPAD, SW), jnp.float32),
            jax.ShapeDtypeStruct((NPAD, PW), jnp.float32),
        ],
    )(r, init1, b1, W2, As2, Ad2)


def _final_body(r_ref, init_ref, b2_ref, out_ref):
    acc = r_ref[0, :, :F] + r_ref[1, :, :F] + init_ref[:, :F]
    s16 = r_ref[0, :, F:] + r_ref[1, :, F:] + init_ref[:, F:]
    s = jnp.concatenate([s16, s16, s16, s16], axis=1)
    o = acc / s + b2_ref[...]
    m = jnp.max(o, axis=1, keepdims=True)
    z = o - m
    lse = jnp.log(jnp.sum(jnp.exp(z), axis=1, keepdims=True))
    out_ref[...] = z - lse


def _final(r, init2, b2):
    return pl.pallas_call(
        _final_body,
        grid=(NROWBLK,),
        in_specs=[
            pl.BlockSpec((2, ROWBLK, PW), lambda i: (0, i, 0)),
            pl.BlockSpec((ROWBLK, PW), lambda i: (i, 0)),
            pl.BlockSpec((1, F), lambda i: (0, 0)),
        ],
        out_specs=pl.BlockSpec((ROWBLK, F), lambda i: (i, 0)),
        out_shape=jax.ShapeDtypeStruct((NPAD, F), jnp.float32),
    )(r, init2, b2)


def _edge_body(ph_hbm, q_hbm, src_hbm, dst_hbm, zero_hbm, o_hbm,
               srcv, dstv, phb0, phb1, qb0, qb1, mb0, mb1,
               racc, gsem0, gsem1, ssem0, ssem1):
    c = lax.axis_index("c")
    t = lax.axis_index("s")
    wid = c * 16 + t
    phb = (phb0, phb1)
    qb = (qb0, qb1)
    mb = (mb0, mb1)
    gsem = (gsem0, gsem1)
    ssem = (ssem0, ssem1)

    pltpu.sync_copy(src_hbm.at[wid], srcv)
    pltpu.sync_copy(dst_hbm.at[wid], dstv)
    pltpu.sync_copy(zero_hbm, racc.at[pl.ds(t * TSLICE, TSLICE)])

    def gather_start(j, par):
        pltpu.async_copy(ph_hbm.at[srcv.at[j]], phb[par], gsem[par])
        pltpu.async_copy(q_hbm.at[dstv.at[j]], qb[par], gsem[par])

    def gather_wait(j, par):
        pltpu.make_async_copy(ph_hbm.at[srcv.at[j]], phb[par], gsem[par]).wait()
        pltpu.make_async_copy(q_hbm.at[dstv.at[j]], qb[par], gsem[par]).wait()

    def scatter_start(j, par):
        pltpu.async_copy(mb[par], racc.at[dstv.at[j]], ssem[par], add=True)

    def scatter_wait(j, par):
        pltpu.make_async_copy(mb[par], racc.at[dstv.at[j]], ssem[par]).wait()

    def compute(par):
        pb = phb[par]
        qq = qb[par]
        ob = mb[par]

        def comp(e, carry2):
            p = pb[e, pl.ds(0, 16)]
            q = qq[e, pl.ds(0, 16)]
            a = p + q
            w = jnp.exp(jnp.where(a >= 0, a, 0.2 * a))
            ob[e, pl.ds(F, 16)] = w
            for j in range(4):
                h = pb[e, pl.ds(SW + j * 16, 16)]
                ob[e, pl.ds(j * 16, 16)] = w * h
            return carry2

        lax.fori_loop(0, EBLK, comp, 0, unroll=2)

    gather_start(0, 0)
    gather_start(1, 1)

    def blk(jj, carry):
        for par in range(2):
            j = 2 * jj + par
            gather_wait(j, par)

            @pl.when(j >= 2)
            def _():
                scatter_wait(j - 2, par)

            compute(par)
            scatter_start(j, par)

            @pl.when(j + 2 < NBLK)
            def _():
                gather_start(j + 2, par)

        return carry

    lax.fori_loop(0, NBLK // 2, blk, 0)
    scatter_wait(NBLK - 2, 0)
    scatter_wait(NBLK - 1, 1)
    plsc.subcore_barrier()

    pltpu.sync_copy(racc.at[pl.ds(t * TSLICE, TSLICE)],
                    o_hbm.at[c, pl.ds(t * TSLICE, TSLICE)])


@functools.cache
def _make_edge_pass():
    return pl.kernel(
        _edge_body,
        out_type=jax.ShapeDtypeStruct((2, NPAD, PW), jnp.float32),
        mesh=plsc.VectorSubcoreMesh(core_axis_name="c", subcore_axis_name="s"),
        compiler_params=pltpu.CompilerParams(use_tc_tiling_on_sc=False),
        scratch_types=[
            pltpu.VMEM((NBLK, EBLK), jnp.int32),
            pltpu.VMEM((NBLK, EBLK), jnp.int32),
            pltpu.VMEM((EBLK, PW), jnp.float32),
            pltpu.VMEM((EBLK, PW), jnp.float32),
            pltpu.VMEM((EBLK, SW), jnp.float32),
            pltpu.VMEM((EBLK, SW), jnp.float32),
            pltpu.VMEM((EBLK, PW), jnp.float32),
            pltpu.VMEM((EBLK, PW), jnp.float32),
            pltpu.VMEM_SHARED((NPAD, PW), jnp.float32),
            pltpu.SemaphoreType.DMA,
            pltpu.SemaphoreType.DMA,
            pltpu.SemaphoreType.DMA,
            pltpu.SemaphoreType.DMA,
        ],
    )


def _edge_pass(ph, q, src, dst, zero):
    return _make_edge_pass()(ph, q, src, dst, zero)


def _expand_heads(att, heads, ch, width):
    a = att.reshape(heads, ch)
    rows = jnp.arange(heads * ch)
    cols = jnp.arange(width)
    m = (a[rows % heads, rows // heads][:, None]
         * ((rows % heads)[:, None] == (cols % heads)[None, :]).astype(a.dtype))
    return m


def kernel(x, edge_index, W1, att_src1, att_dst1, b1, W2, att_src2,
           att_dst2, b2):
    perm = (jnp.arange(F) % 8) * 8 + jnp.arange(F) // 8
    W1p = W1[:, perm]
    b1p = b1[perm]
    W2p = W2[perm, :]

    xp = jnp.zeros((NPAD, D_IN), jnp.float32).at[:N].set(x)
    As1 = _expand_heads(att_src1, 8, 8, SW)
    Ad1 = _expand_heads(att_dst1, 8, 8, SW)
    As2 = _expand_heads(att_src2, 1, F, SW)
    Ad2 = _expand_heads(att_dst2, 1, F, SW)

    pad_ids = jnp.full((EPAD - E,), N, dtype=jnp.int32)
    src = jnp.concatenate([edge_index[0], pad_ids]).reshape(NW, NBLK, EBLK)
    dst = jnp.concatenate([edge_index[1], pad_ids]).reshape(NW, NBLK, EBLK)
    zero = jnp.zeros((TSLICE, PW), jnp.float32)

    ph1, q1, init1 = _prep1(xp, W1p, As1, Ad1)
    r1 = _edge_pass(ph1, q1, src, dst, zero)
    ph2, q2, init2 = _mid(r1, init1, b1p.reshape(1, F), W2p, As2, Ad2)
    r2 = _edge_pass(ph2, q2, src, dst, zero)
    out = _final(r2, init2, b2.reshape(1, F))
    return out[:N]
